# baseline (speedup 1.0000x reference)
import functools

import jax
import jax.numpy as jnp
from jax import lax
from jax.experimental import pallas as pl
from jax.experimental.pallas import tpu as pltpu
from jax.experimental.pallas import tpu_sc as plsc

NC, NS, L = 2, 16, 16
C = 19
KK = 9
H = 64
W = 64
HB = 32
B = 2


def _vgather(v, idx):
    return lax.gather(
        v, idx[:, None],
        lax.GatherDimensionNumbers(offset_dims=(), collapsed_slice_dims=(0,),
                                   start_index_map=(0,)),
        (1,), mode=lax.GatherScatterMode.PROMISE_IN_BOUNDS)


def _sc_body(seg_hbm, ori_hbm, part_hbm, state_hbm,
             logits_v, prob_v, pos_v, ori_v, state_v, out_v,
             seg_sem, ori_sem):
    core = lax.axis_index("c")
    sub = lax.axis_index("s")
    wid = core * NS + sub
    b = core
    jb0 = sub * 2
    h0 = jb0 * 2
    hstart = pl.multiple_of(jnp.clip(8 * ((h0 - 1) // 8), 0, H - 16), 8)
    jb8 = pl.multiple_of(8 * (jb0 // 8), 8)
    lr0 = jb0 - jb8

    iota = lax.broadcasted_iota(jnp.int32, (L,), 0)
    zero16 = jnp.zeros((L,), jnp.float32)

    seg_dma = pltpu.async_copy(
        seg_hbm.at[b, :, pl.ds(hstart, 16), :], logits_v, seg_sem)
    ori_dma = pltpu.async_copy(
        ori_hbm.at[b, :, pl.ds(jb8, 8), :], ori_v, ori_sem)
    seg_dma.wait()

    def _softmax_row(lr, _):
        h = h0 - 1 + lr
        @pl.when((h >= 0) & (h < H))
        def _():
            src = h - hstart
            for c in range(C):
                prob_v[pl.ds(c * 480 + lr * 80, L)] = zero16
                prob_v[pl.ds(c * 480 + lr * 80 + 50, L)] = zero16
            for chk in range(4):
                w0 = chk * L
                vals = [logits_v[c, src, pl.ds(w0, L)] for c in range(C)]
                m = vals[0]
                for c in range(1, C):
                    m = jnp.maximum(m, vals[c])
                es = [jnp.exp(v - m) for v in vals]
                s = es[0]
                for c in range(1, C):
                    s = s + es[c]
                inv = 1.0 / s
                for c in range(C):
                    prob_v[pl.ds(c * 480 + lr * 80 + w0 + 1, L)] = es[c] * inv
        @pl.when((h < 0) | (h >= H))
        def _():
            for c in range(C):
                for k in range(5):
                    prob_v[pl.ds(c * 480 + lr * 80 + k * L, L)] = zero16
        return 0
    lax.fori_loop(0, 6, _softmax_row, 0)

    def _pos_row(r, _):
        def _pos_chunk(chk, _2):
            w0 = chk * L
            center = [prob_v[pl.ds(c * 480 + (r + 1) * 80 + w0 + 1, L)]
                      for c in range(C)]
            for kk in range(KK):
                dy, dx = kk // 3, kk % 3
                acc = zero16
                for c in range(C):
                    acc = acc + center[c] * prob_v[
                        pl.ds(c * 480 + (r + dy) * 80 + w0 + dx, L)]
                pos_v[pl.ds(kk * 256 + r * 64 + w0, L)] = acc
            return 0
        lax.fori_loop(0, 4, _pos_chunk, 0)
        return 0
    lax.fori_loop(0, 4, _pos_row, 0)

    ori_dma.wait()
    neg_inf = jnp.full((L,), -jnp.inf, jnp.float32)
    pos_inf = jnp.full((L,), jnp.inf, jnp.float32)

    lo8 = iota < 8
    idx_e = jnp.where(lo8, iota * 2, iota * 2 - L)
    idx_o = idx_e + 1

    def _block(i, carry):
        accp, accn, accc = carry
        hb = i // 2
        chk = i % 2
        w0 = chk * L
        wbv = iota + w0
        r2 = hb * 2
        ha = h0 + r2
        Pv, Nv = [], []
        for kk in range(KK):
            dy, dx = kk // 3, kk % 3
            base = kk * 256 + r2 * 64 + w0 * 2
            t0 = pos_v[pl.ds(base, L)] + pos_v[pl.ds(base + 64, L)]
            t1 = pos_v[pl.ds(base + L, L)] + pos_v[pl.ds(base + 64 + L, L)]
            g = (jnp.where(lo8, _vgather(t0, idx_e), _vgather(t1, idx_e))
                 + jnp.where(lo8, _vgather(t0, idx_o), _vgather(t1, idx_o)))
            rc = (((ha + dy - 1 >= 0) & (ha + dy - 1 < H)).astype(jnp.float32)
                  + ((ha + dy >= 0) & (ha + dy < H)).astype(jnp.float32))
            if dx == 0:
                colc = jnp.where(wbv == 0, 1.0, 2.0)
            elif dx == 2:
                colc = jnp.where(wbv == HB - 1, 1.0, 2.0)
            else:
                colc = jnp.full((L,), 2.0, jnp.float32)
            Pv.append(g)
            Nv.append(rc * colc - g)
        sf = [ori_v[kk, lr0 + hb, pl.ds(w0, L)] for kk in range(KK)]
        maskf = jnp.where(sf[0] > 0, 1.0, 0.0)
        one16 = zero16 + 1.0

        def select(payload, nsel, largest):
            alive = [one16 for _ in range(KK)]
            tot = zero16
            for _s in range(nsel):
                cur = neg_inf if largest else pos_inf
                curp = zero16
                for kk in range(KK):
                    if largest:
                        better = jnp.where(sf[kk] > cur, 1.0, 0.0)
                    else:
                        better = jnp.where(sf[kk] < cur, 1.0, 0.0)
                    take = alive[kk] * better
                    cur = jnp.where(take > 0.5, sf[kk], cur)
                    curp = jnp.where(take > 0.5, payload[kk], curp)
                fm = zero16
                for kk in range(KK):
                    eq = jnp.where(sf[kk] == cur, 1.0, 0.0)
                    selm = alive[kk] * eq * (1.0 - fm)
                    fm = fm + selm
                    alive[kk] = alive[kk] - selm
                if largest:
                    tot = tot - cur * curp
                else:
                    tot = tot - (1.0 - cur) * curp
            return tot
        possum = select(Pv, 5, True)
        negsum = select(Nv, 4, False)
        accp = accp + maskf * possum
        accn = accn + maskf * negsum
        accc = accc + maskf
        return accp, accn, accc

    accp, accn, accc = lax.fori_loop(0, 4, _block, (zero16, zero16, zero16))

    for hb in range(2):
        for chk in range(2):
            w0 = chk * L
            s = zero16
            for kk in range(KK):
                s = s + ori_v[kk, lr0 + hb, pl.ds(w0, L)]
            state_v[pl.ds(hb * 32 + w0, L)] = s * (1.0 / KK)
    soff = pl.multiple_of(b * (HB * HB) + jb0 * HB, 8)
    pltpu.sync_copy(state_v, state_hbm.at[pl.ds(soff, 2 * HB)])

    out_v[pl.ds(0, L)] = accp
    out_v[pl.ds(L, L)] = accn
    out_v[pl.ds(2 * L, L)] = accc
    nw = NC * NS
    for f in range(3):
        pltpu.sync_copy(out_v.at[pl.ds(f * L, L)],
                        part_hbm.at[pl.ds(f * nw * L + wid * L, L)])


def _tc_combine(part_ref, out_ref):
    nw = NC * NS
    sp = jnp.sum(part_ref[pl.ds(0, nw * L)])
    sn = jnp.sum(part_ref[pl.ds(nw * L, nw * L)])
    c32 = jnp.sum(part_ref[pl.ds(2 * nw * L, nw * L)])
    cnt_pos = jnp.maximum(c32 * 4.0 * 5.0, 1.0)
    cnt_neg = jnp.maximum(c32 * 4.0 * 4.0, 1.0)
    lp = sp / cnt_pos * 0.5
    ln = sn / cnt_neg
    r2 = lax.broadcasted_iota(jnp.int32, (8, 128), 0)
    c2 = lax.broadcasted_iota(jnp.int32, (8, 128), 1)
    out_ref[...] = jnp.where((r2 == 0) & (c2 == 0), lp,
                             jnp.where((r2 == 0) & (c2 == 1), ln, 0.0))


@functools.partial(
    pl.kernel,
    mesh=plsc.VectorSubcoreMesh(core_axis_name="c", subcore_axis_name="s"),
    out_type=[
        jax.ShapeDtypeStruct((NC * NS * 3 * L,), jnp.float32),
        jax.ShapeDtypeStruct((B * HB * HB,), jnp.float32),
    ],
    scratch_types=[
        pltpu.VMEM((C, 16, W), jnp.float32),
        pltpu.VMEM((C * 6 * 80,), jnp.float32),
        pltpu.VMEM((KK * 4 * W,), jnp.float32),
        pltpu.VMEM((KK, 8, HB), jnp.float32),
        pltpu.VMEM((2 * HB,), jnp.float32),
        pltpu.VMEM((3 * L,), jnp.float32),
        pltpu.SemaphoreType.DMA,
        pltpu.SemaphoreType.DMA,
    ],
)
def _sc_kernel(seg_hbm, ori_hbm, part_hbm, state_hbm, *scratch):
    _sc_body(seg_hbm, ori_hbm, part_hbm, state_hbm, *scratch)


def kernel(ori_sim_feats_list, seg_logits):
    partials, state = _sc_kernel(seg_logits, ori_sim_feats_list)
    losses = pl.pallas_call(
        _tc_combine,
        out_shape=jax.ShapeDtypeStruct((8, 128), jnp.float32),
    )(partials)
    return losses[0, 0], losses[0, 1], state.reshape(B, HB, HB)


# speedup vs baseline: 1.0430x; 1.0430x over previous
import functools

import jax
import jax.numpy as jnp
from jax import lax
from jax.experimental import pallas as pl
from jax.experimental.pallas import tpu as pltpu
from jax.experimental.pallas import tpu_sc as plsc

NC, NS, L = 2, 16, 16
C = 19
KK = 9
H = 64
W = 64
HB = 32
B = 2


def _vgather(v, idx):
    return lax.gather(
        v, idx[:, None],
        lax.GatherDimensionNumbers(offset_dims=(), collapsed_slice_dims=(0,),
                                   start_index_map=(0,)),
        (1,), mode=lax.GatherScatterMode.PROMISE_IN_BOUNDS)


def _sc_body(seg_hbm, ori_hbm, part_hbm, state_hbm,
             logits_v, prob_v, pos_v, ori_v, state_v, out_v,
             seg_sem, ori_sem):
    wid = lax.axis_index("s") * NC + lax.axis_index("c")
    b = wid // NS
    jb0 = (wid % NS) * 2
    h0 = jb0 * 2
    hstart = jnp.clip(h0 - 1, 0, H - 6)

    iota = lax.broadcasted_iota(jnp.int32, (L,), 0)
    zero16 = jnp.zeros((L,), jnp.float32)

    seg_dmas = []
    for c in range(C):
        src = pl.multiple_of((b * C + c) * (H * W) + hstart * W, 8)
        seg_dmas.append(pltpu.async_copy(
            seg_hbm.at[pl.ds(src, 6 * W)],
            logits_v.at[pl.ds(c * 6 * W, 6 * W)], seg_sem))
    ori_dmas = []
    for kk in range(KK):
        src = pl.multiple_of((b * KK + kk) * (HB * HB) + jb0 * HB, 8)
        ori_dmas.append(pltpu.async_copy(
            ori_hbm.at[pl.ds(src, 2 * HB)],
            ori_v.at[pl.ds(kk * 2 * HB, 2 * HB)], ori_sem))
    for d in seg_dmas:
        d.wait()

    def _softmax_row(lr, _):
        h = h0 - 1 + lr
        @pl.when((h >= 0) & (h < H))
        def _():
            src = h - hstart

            def _zpad(c, _2):
                prob_v[pl.ds(c * 480 + lr * 80, L)] = zero16
                prob_v[pl.ds(c * 480 + lr * 80 + 50, L)] = zero16
                return 0
            lax.fori_loop(0, C, _zpad, 0)

            def _chk(chk, _2):
                w0 = chk * L
                vals = [logits_v[pl.ds(c * 6 * W + src * W + w0, L)]
                        for c in range(C)]
                m = vals[0]
                for c in range(1, C):
                    m = jnp.maximum(m, vals[c])
                es = [jnp.exp(v - m) for v in vals]
                s = es[0]
                for c in range(1, C):
                    s = s + es[c]
                inv = 1.0 / s
                for c in range(C):
                    prob_v[pl.ds(c * 480 + lr * 80 + w0 + 1, L)] = es[c] * inv
                return 0
            lax.fori_loop(0, 4, _chk, 0)
        @pl.when((h < 0) | (h >= H))
        def _():
            def _zrow(i, _2):
                prob_v[pl.ds((i // 5) * 480 + lr * 80 + (i % 5) * L, L)] = (
                    zero16)
                return 0
            lax.fori_loop(0, C * 5, _zrow, 0)
        return 0
    lax.fori_loop(0, 6, _softmax_row, 0)

    def _pos_row(r, _):
        def _pos_chunk(chk, _2):
            w0 = chk * L
            center = [prob_v[pl.ds(c * 480 + (r + 1) * 80 + w0 + 1, L)]
                      for c in range(C)]

            def _kk(kk, _3):
                dy, dx = kk // 3, kk % 3
                rowoff = (r + dy) * 80 + w0 + dx
                acc = zero16
                for c in range(C):
                    acc = acc + center[c] * prob_v[pl.ds(c * 480 + rowoff, L)]
                pos_v[pl.ds(kk * 256 + r * 64 + w0, L)] = acc
                return 0
            lax.fori_loop(0, KK, _kk, 0, unroll=3)
            return 0
        lax.fori_loop(0, 4, _pos_chunk, 0)
        return 0
    lax.fori_loop(0, 4, _pos_row, 0)

    for d in ori_dmas:
        d.wait()
    neg_inf = jnp.full((L,), -jnp.inf, jnp.float32)
    pos_inf = jnp.full((L,), jnp.inf, jnp.float32)

    lo8 = iota < 8
    idx_e = jnp.where(lo8, iota * 2, iota * 2 - L)
    idx_o = idx_e + 1

    def _block(i, carry):
        accp, accn, accc = carry
        hb = i // 2
        chk = i % 2
        w0 = chk * L
        wbv = iota + w0
        r2 = hb * 2
        ha = h0 + r2
        Pv, Nv = [], []
        for kk in range(KK):
            dy, dx = kk // 3, kk % 3
            base = kk * 256 + r2 * 64 + w0 * 2
            t0 = pos_v[pl.ds(base, L)] + pos_v[pl.ds(base + 64, L)]
            t1 = pos_v[pl.ds(base + L, L)] + pos_v[pl.ds(base + 64 + L, L)]
            g = (jnp.where(lo8, _vgather(t0, idx_e), _vgather(t1, idx_e))
                 + jnp.where(lo8, _vgather(t0, idx_o), _vgather(t1, idx_o)))
            rc = (((ha + dy - 1 >= 0) & (ha + dy - 1 < H)).astype(jnp.float32)
                  + ((ha + dy >= 0) & (ha + dy < H)).astype(jnp.float32))
            if dx == 0:
                colc = jnp.where(wbv == 0, 1.0, 2.0)
            elif dx == 2:
                colc = jnp.where(wbv == HB - 1, 1.0, 2.0)
            else:
                colc = jnp.full((L,), 2.0, jnp.float32)
            Pv.append(g)
            Nv.append(rc * colc - g)
        sf = [ori_v[pl.ds(kk * 64 + hb * 32 + w0, L)] for kk in range(KK)]
        maskf = jnp.where(sf[0] > 0, 1.0, 0.0)
        one16 = zero16 + 1.0

        def select(payload, nsel, largest):
            alive = [one16 for _ in range(KK)]
            tot = zero16
            for _s in range(nsel):
                cur = neg_inf if largest else pos_inf
                curp = zero16
                for kk in range(KK):
                    if largest:
                        better = jnp.where(sf[kk] > cur, 1.0, 0.0)
                    else:
                        better = jnp.where(sf[kk] < cur, 1.0, 0.0)
                    take = alive[kk] * better
                    cur = jnp.where(take > 0.5, sf[kk], cur)
                    curp = jnp.where(take > 0.5, payload[kk], curp)
                fm = zero16
                for kk in range(KK):
                    eq = jnp.where(sf[kk] == cur, 1.0, 0.0)
                    selm = alive[kk] * eq * (1.0 - fm)
                    fm = fm + selm
                    alive[kk] = alive[kk] - selm
                if largest:
                    tot = tot - cur * curp
                else:
                    tot = tot - (1.0 - cur) * curp
            return tot
        possum = select(Pv, 5, True)
        negsum = select(Nv, 4, False)
        accp = accp + maskf * possum
        accn = accn + maskf * negsum
        accc = accc + maskf
        return accp, accn, accc

    accp, accn, accc = lax.fori_loop(0, 4, _block, (zero16, zero16, zero16))

    def _st(i, _):
        off = i * L
        s = zero16
        for kk in range(KK):
            s = s + ori_v[pl.ds(kk * 64 + off, L)]
        state_v[pl.ds(off, L)] = s * (1.0 / KK)
        return 0
    lax.fori_loop(0, 4, _st, 0)
    soff = pl.multiple_of(b * (HB * HB) + jb0 * HB, 8)
    pltpu.sync_copy(state_v, state_hbm.at[pl.ds(soff, 2 * HB)])

    out_v[pl.ds(0, L)] = accp
    out_v[pl.ds(L, L)] = accn
    out_v[pl.ds(2 * L, L)] = accc
    nw = NC * NS
    for f in range(3):
        pltpu.sync_copy(out_v.at[pl.ds(f * L, L)],
                        part_hbm.at[pl.ds(f * nw * L + wid * L, L)])


def _tc_combine(part_ref, out_ref):
    nw = NC * NS
    sp = jnp.sum(part_ref[pl.ds(0, nw * L)])
    sn = jnp.sum(part_ref[pl.ds(nw * L, nw * L)])
    c32 = jnp.sum(part_ref[pl.ds(2 * nw * L, nw * L)])
    cnt_pos = jnp.maximum(c32 * 4.0 * 5.0, 1.0)
    cnt_neg = jnp.maximum(c32 * 4.0 * 4.0, 1.0)
    lp = sp / cnt_pos * 0.5
    ln = sn / cnt_neg
    r2 = lax.broadcasted_iota(jnp.int32, (8, 128), 0)
    c2 = lax.broadcasted_iota(jnp.int32, (8, 128), 1)
    out_ref[...] = jnp.where((r2 == 0) & (c2 == 0), lp,
                             jnp.where((r2 == 0) & (c2 == 1), ln, 0.0))


@functools.partial(
    pl.kernel,
    mesh=plsc.VectorSubcoreMesh(core_axis_name="c", subcore_axis_name="s"),
    out_type=[
        jax.ShapeDtypeStruct((NC * NS * 3 * L,), jnp.float32),
        jax.ShapeDtypeStruct((B * HB * HB,), jnp.float32),
    ],
    scratch_types=[
        pltpu.VMEM((C * 6 * W,), jnp.float32),
        pltpu.VMEM((C * 6 * 80,), jnp.float32),
        pltpu.VMEM((KK * 4 * W,), jnp.float32),
        pltpu.VMEM((KK * 2 * HB,), jnp.float32),
        pltpu.VMEM((2 * HB,), jnp.float32),
        pltpu.VMEM((3 * L,), jnp.float32),
        pltpu.SemaphoreType.DMA,
        pltpu.SemaphoreType.DMA,
    ],
)
def _sc_kernel(seg_hbm, ori_hbm, part_hbm, state_hbm, *scratch):
    _sc_body(seg_hbm, ori_hbm, part_hbm, state_hbm, *scratch)


def kernel(ori_sim_feats_list, seg_logits):
    partials, state = _sc_kernel(seg_logits.reshape(-1),
                                 ori_sim_feats_list.reshape(-1))
    losses = pl.pallas_call(
        _tc_combine,
        out_shape=jax.ShapeDtypeStruct((8, 128), jnp.float32),
    )(partials)
    return losses[0, 0], losses[0, 1], state.reshape(B, HB, HB)


# speedup vs baseline: 1.2439x; 1.1927x over previous
import functools

import jax
import jax.numpy as jnp
from jax import lax
from jax.experimental import pallas as pl
from jax.experimental.pallas import tpu as pltpu
from jax.experimental.pallas import tpu_sc as plsc

NC, NS, L = 2, 16, 16
C = 19
KK = 9
H = 64
W = 64
HB = 32
B = 2


def _vgather(v, idx):
    return lax.gather(
        v, idx[:, None],
        lax.GatherDimensionNumbers(offset_dims=(), collapsed_slice_dims=(0,),
                                   start_index_map=(0,)),
        (1,), mode=lax.GatherScatterMode.PROMISE_IN_BOUNDS)


def _sc_body(seg_hbm, ori_hbm, part_hbm, state_hbm,
             logits_v, prob_v, pos_v, ori_v, state_v, out_v,
             seg_sem, ori_sem):
    wid = lax.axis_index("s") * NC + lax.axis_index("c")
    b = wid // NS
    jb0 = (wid % NS) * 2
    h0 = jb0 * 2
    hstart = jnp.clip(h0 - 1, 0, H - 6)

    iota = lax.broadcasted_iota(jnp.int32, (L,), 0)
    zero16 = jnp.zeros((L,), jnp.float32)

    seg_dmas = []
    for c in range(C):
        src = pl.multiple_of((b * C + c) * (H * W) + hstart * W, 8)
        seg_dmas.append(pltpu.async_copy(
            seg_hbm.at[pl.ds(src, 6 * W)],
            logits_v.at[pl.ds(c * 6 * W, 6 * W)], seg_sem))
    ori_dmas = []
    for kk in range(KK):
        src = pl.multiple_of((b * KK + kk) * (HB * HB) + jb0 * HB, 8)
        ori_dmas.append(pltpu.async_copy(
            ori_hbm.at[pl.ds(src, 2 * HB)],
            ori_v.at[pl.ds(kk * 2 * HB, 2 * HB)], ori_sem))
    for d in seg_dmas:
        d.wait()

    def _softmax_row(lr, _):
        h = h0 - 1 + lr
        @pl.when((h >= 0) & (h < H))
        def _():
            src = h - hstart

            def _zpad(c, _2):
                prob_v[pl.ds(c * 480 + lr * 80, L)] = zero16
                prob_v[pl.ds(c * 480 + lr * 80 + 50, L)] = zero16
                return 0
            lax.fori_loop(0, C, _zpad, 0)

            def _chk(chk, _2):
                w0 = chk * L
                vals = [logits_v[pl.ds(c * 6 * W + src * W + w0, L)]
                        for c in range(C)]
                m = vals[0]
                for c in range(1, C):
                    m = jnp.maximum(m, vals[c])
                es = [jnp.exp(v - m) for v in vals]
                s = es[0]
                for c in range(1, C):
                    s = s + es[c]
                inv = 1.0 / s
                for c in range(C):
                    prob_v[pl.ds(c * 480 + lr * 80 + w0 + 1, L)] = es[c] * inv
                return 0
            lax.fori_loop(0, 4, _chk, 0, unroll=2)
        @pl.when((h < 0) | (h >= H))
        def _():
            def _zrow(i, _2):
                prob_v[pl.ds((i // 5) * 480 + lr * 80 + (i % 5) * L, L)] = (
                    zero16)
                return 0
            lax.fori_loop(0, C * 5, _zrow, 0)
        return 0
    lax.fori_loop(0, 6, _softmax_row, 0)

    def _pos_row(rr, _):
        def _zp(j, _2):
            pos_v[pl.ds(j * 400 + rr * 80, L)] = zero16
            pos_v[pl.ds(j * 400 + rr * 80 + 50, L)] = zero16
            return 0
        lax.fori_loop(0, 5, _zp, 0)

        def _pos_chunk(chk, _2):
            w0 = chk * L
            center = [prob_v[pl.ds(c * 480 + rr * 80 + w0 + 1, L)]
                      for c in range(C)]

            def _kk(j, _3):
                dy, dx = (j + 4) // 3, (j + 4) % 3
                rowoff = (rr - 1 + dy) * 80 + w0 + dx
                acc = zero16
                for c in range(C):
                    acc = acc + center[c] * prob_v[pl.ds(c * 480 + rowoff, L)]
                pos_v[pl.ds(j * 400 + rr * 80 + w0 + 1, L)] = acc
                return 0
            lax.fori_loop(0, 5, _kk, 0, unroll=5)
            return 0
        lax.fori_loop(0, 4, _pos_chunk, 0, unroll=2)
        return 0
    lax.fori_loop(0, 5, _pos_row, 0)

    for d in ori_dmas:
        d.wait()
    neg_inf = jnp.full((L,), -jnp.inf, jnp.float32)
    pos_inf = jnp.full((L,), jnp.inf, jnp.float32)

    lo8 = iota < 8
    idx_e = jnp.where(lo8, iota * 2, iota * 2 - L)
    idx_o = idx_e + 1

    def _block(i, carry):
        accp, accn, accc = carry
        hb = i // 2
        chk = i % 2
        w0 = chk * L
        wbv = iota + w0
        r2 = hb * 2
        ha = h0 + r2
        Pv, Nv = [], []
        for kk in range(KK):
            dy, dx = kk // 3, kk % 3
            if kk >= 4:
                base = (kk - 4) * 400 + (r2 + 1) * 80 + w0 * 2 + 1
            else:
                base = (4 - kk) * 400 + (r2 + dy) * 80 + w0 * 2 + dx
            t0 = pos_v[pl.ds(base, L)] + pos_v[pl.ds(base + 80, L)]
            t1 = pos_v[pl.ds(base + L, L)] + pos_v[pl.ds(base + 80 + L, L)]
            g = (jnp.where(lo8, _vgather(t0, idx_e), _vgather(t1, idx_e))
                 + jnp.where(lo8, _vgather(t0, idx_o), _vgather(t1, idx_o)))
            rc = (((ha + dy - 1 >= 0) & (ha + dy - 1 < H)).astype(jnp.float32)
                  + ((ha + dy >= 0) & (ha + dy < H)).astype(jnp.float32))
            if dx == 0:
                colc = jnp.where(wbv == 0, 1.0, 2.0)
            elif dx == 2:
                colc = jnp.where(wbv == HB - 1, 1.0, 2.0)
            else:
                colc = jnp.full((L,), 2.0, jnp.float32)
            Pv.append(g)
            Nv.append(rc * colc - g)
        sf = [ori_v[pl.ds(kk * 64 + hb * 32 + w0, L)] for kk in range(KK)]
        maskf = jnp.where(sf[0] > 0, 1.0, 0.0)
        one16 = zero16 + 1.0

        def select(payload, nsel, largest):
            alive = [one16 for _ in range(KK)]
            tot = zero16
            for _s in range(nsel):
                cur = neg_inf if largest else pos_inf
                curp = zero16
                for kk in range(KK):
                    if largest:
                        better = jnp.where(sf[kk] > cur, 1.0, 0.0)
                    else:
                        better = jnp.where(sf[kk] < cur, 1.0, 0.0)
                    take = alive[kk] * better
                    cur = jnp.where(take > 0.5, sf[kk], cur)
                    curp = jnp.where(take > 0.5, payload[kk], curp)
                fm = zero16
                for kk in range(KK):
                    eq = jnp.where(sf[kk] == cur, 1.0, 0.0)
                    selm = alive[kk] * eq * (1.0 - fm)
                    fm = fm + selm
                    alive[kk] = alive[kk] - selm
                if largest:
                    tot = tot - cur * curp
                else:
                    tot = tot - (1.0 - cur) * curp
            return tot
        possum = select(Pv, 5, True)
        negsum = select(Nv, 4, False)
        accp = accp + maskf * possum
        accn = accn + maskf * negsum
        accc = accc + maskf
        return accp, accn, accc

    accp, accn, accc = lax.fori_loop(0, 4, _block, (zero16, zero16, zero16))

    def _st(i, _):
        off = i * L
        s = zero16
        for kk in range(KK):
            s = s + ori_v[pl.ds(kk * 64 + off, L)]
        state_v[pl.ds(off, L)] = s * (1.0 / KK)
        return 0
    lax.fori_loop(0, 4, _st, 0)
    soff = pl.multiple_of(b * (HB * HB) + jb0 * HB, 8)
    pltpu.sync_copy(state_v, state_hbm.at[pl.ds(soff, 2 * HB)])

    out_v[pl.ds(0, L)] = accp
    out_v[pl.ds(L, L)] = accn
    out_v[pl.ds(2 * L, L)] = accc
    nw = NC * NS
    for f in range(3):
        pltpu.sync_copy(out_v.at[pl.ds(f * L, L)],
                        part_hbm.at[pl.ds(f * nw * L + wid * L, L)])


def _tc_combine(part_ref, state_ref, lp_ref, ln_ref, state_out_ref):
    nw = NC * NS
    sp = jnp.sum(part_ref[pl.ds(0, nw * L)])
    sn = jnp.sum(part_ref[pl.ds(nw * L, nw * L)])
    c32 = jnp.sum(part_ref[pl.ds(2 * nw * L, nw * L)])
    cnt_pos = jnp.maximum(c32 * 4.0 * 5.0, 1.0)
    cnt_neg = jnp.maximum(c32 * 4.0 * 4.0, 1.0)
    lp_ref[0] = sp / cnt_pos * 0.5
    ln_ref[0] = sn / cnt_neg
    for b in range(B):
        for r in range(HB):
            flat = b * HB * HB + r * HB
            state_out_ref[b, r, :] = state_ref[flat // 128,
                                               pl.ds(flat % 128, HB)]


@functools.partial(
    pl.kernel,
    mesh=plsc.VectorSubcoreMesh(core_axis_name="c", subcore_axis_name="s"),
    out_type=[
        jax.ShapeDtypeStruct((NC * NS * 3 * L,), jnp.float32),
        jax.ShapeDtypeStruct((B * HB * HB,), jnp.float32),
    ],
    scratch_types=[
        pltpu.VMEM((C * 6 * W,), jnp.float32),
        pltpu.VMEM((C * 6 * 80,), jnp.float32),
        pltpu.VMEM((5 * 5 * 80,), jnp.float32),
        pltpu.VMEM((KK * 2 * HB,), jnp.float32),
        pltpu.VMEM((2 * HB,), jnp.float32),
        pltpu.VMEM((3 * L,), jnp.float32),
        pltpu.SemaphoreType.DMA,
        pltpu.SemaphoreType.DMA,
    ],
)
def _sc_kernel(seg_hbm, ori_hbm, part_hbm, state_hbm, *scratch):
    _sc_body(seg_hbm, ori_hbm, part_hbm, state_hbm, *scratch)


def kernel(ori_sim_feats_list, seg_logits):
    partials, state = _sc_kernel(seg_logits.reshape(-1),
                                 ori_sim_feats_list.reshape(-1))
    lp, ln, state3 = pl.pallas_call(
        _tc_combine,
        out_shape=[jax.ShapeDtypeStruct((1,), jnp.float32),
                   jax.ShapeDtypeStruct((1,), jnp.float32),
                   jax.ShapeDtypeStruct((B, HB, HB), jnp.float32)],
        out_specs=[pl.BlockSpec(memory_space=pltpu.SMEM),
                   pl.BlockSpec(memory_space=pltpu.SMEM),
                   pl.BlockSpec(memory_space=pltpu.VMEM)],
    )(partials, state.reshape(16, 128))
    return lp.reshape(()), ln.reshape(()), state3


# speedup vs baseline: 1.2444x; 1.0004x over previous
import functools

import jax
import jax.numpy as jnp
from jax import lax
from jax.experimental import pallas as pl
from jax.experimental.pallas import tpu as pltpu
from jax.experimental.pallas import tpu_sc as plsc

NC, NS, L = 2, 16, 16
C = 19
KK = 9
H = 64
W = 64
HB = 32
B = 2


def _vgather(v, idx):
    return lax.gather(
        v, idx[:, None],
        lax.GatherDimensionNumbers(offset_dims=(), collapsed_slice_dims=(0,),
                                   start_index_map=(0,)),
        (1,), mode=lax.GatherScatterMode.PROMISE_IN_BOUNDS)


def _sc_body(seg_hbm, ori_hbm, part_hbm, state_hbm,
             logits_v, prob_v, pos_v, ori_v, state_v, out_v,
             seg_sem, ori_sem):
    wid = lax.axis_index("s") * NC + lax.axis_index("c")
    b = wid // NS
    jb0 = (wid % NS) * 2
    h0 = jb0 * 2
    hstart = jnp.clip(h0 - 1, 0, H - 6)

    iota = lax.broadcasted_iota(jnp.int32, (L,), 0)
    zero16 = jnp.zeros((L,), jnp.float32)

    seg_dmas = []
    for c in range(C):
        src = pl.multiple_of((b * C + c) * (H * W) + hstart * W, 8)
        seg_dmas.append(pltpu.async_copy(
            seg_hbm.at[pl.ds(src, 6 * W)],
            logits_v.at[pl.ds(c * 6 * W, 6 * W)], seg_sem))
    ori_dmas = []
    for kk in range(KK):
        src = pl.multiple_of((b * KK + kk) * (HB * HB) + jb0 * HB, 8)
        ori_dmas.append(pltpu.async_copy(
            ori_hbm.at[pl.ds(src, 2 * HB)],
            ori_v.at[pl.ds(kk * 2 * HB, 2 * HB)], ori_sem))
    for d in seg_dmas:
        d.wait()

    def _softmax_row(lr, _):
        h = h0 - 1 + lr
        @pl.when((h >= 0) & (h < H))
        def _():
            src = h - hstart

            def _zpad(c, _2):
                prob_v[pl.ds(c * 480 + lr * 80, L)] = zero16
                prob_v[pl.ds(c * 480 + lr * 80 + 50, L)] = zero16
                return 0
            lax.fori_loop(0, C, _zpad, 0)

            def _chk(chk, _2):
                w0 = chk * L
                vals = [logits_v[pl.ds(c * 6 * W + src * W + w0, L)]
                        for c in range(C)]
                m = vals[0]
                for c in range(1, C):
                    m = jnp.maximum(m, vals[c])
                es = [jnp.exp(v - m) for v in vals]
                s = es[0]
                for c in range(1, C):
                    s = s + es[c]
                inv = 1.0 / s
                for c in range(C):
                    prob_v[pl.ds(c * 480 + lr * 80 + w0 + 1, L)] = es[c] * inv
                return 0
            lax.fori_loop(0, 4, _chk, 0, unroll=2)
        @pl.when((h < 0) | (h >= H))
        def _():
            def _zrow(i, _2):
                prob_v[pl.ds((i // 5) * 480 + lr * 80 + (i % 5) * L, L)] = (
                    zero16)
                return 0
            lax.fori_loop(0, C * 5, _zrow, 0)
        return 0
    lax.fori_loop(0, 6, _softmax_row, 0)

    def _pos_row(rr, _):
        def _zp(j, _2):
            pos_v[pl.ds(j * 400 + rr * 80, L)] = zero16
            pos_v[pl.ds(j * 400 + rr * 80 + 50, L)] = zero16
            return 0
        lax.fori_loop(0, 5, _zp, 0)

        def _pos_chunk(chk, _2):
            w0 = chk * L
            center = [prob_v[pl.ds(c * 480 + rr * 80 + w0 + 1, L)]
                      for c in range(C)]

            def _kk(j, _3):
                dy, dx = (j + 4) // 3, (j + 4) % 3
                rowoff = (rr - 1 + dy) * 80 + w0 + dx
                acc = zero16
                for c in range(C):
                    acc = acc + center[c] * prob_v[pl.ds(c * 480 + rowoff, L)]
                pos_v[pl.ds(j * 400 + rr * 80 + w0 + 1, L)] = acc
                return 0
            lax.fori_loop(0, 5, _kk, 0, unroll=5)
            return 0
        lax.fori_loop(0, 4, _pos_chunk, 0)
        return 0
    lax.fori_loop(0, 5, _pos_row, 0)

    for d in ori_dmas:
        d.wait()
    neg_inf = jnp.full((L,), -jnp.inf, jnp.float32)
    pos_inf = jnp.full((L,), jnp.inf, jnp.float32)

    lo8 = iota < 8
    idx_e = jnp.where(lo8, iota * 2, iota * 2 - L)
    idx_o = idx_e + 1

    def _block(i, carry):
        accp, accn, accc = carry
        hb = i // 2
        chk = i % 2
        w0 = chk * L
        wbv = iota + w0
        r2 = hb * 2
        ha = h0 + r2
        Pv, Nv = [], []
        for kk in range(KK):
            dy, dx = kk // 3, kk % 3
            if kk >= 4:
                base = (kk - 4) * 400 + (r2 + 1) * 80 + w0 * 2 + 1
            else:
                base = (4 - kk) * 400 + (r2 + dy) * 80 + w0 * 2 + dx
            t0 = pos_v[pl.ds(base, L)] + pos_v[pl.ds(base + 80, L)]
            t1 = pos_v[pl.ds(base + L, L)] + pos_v[pl.ds(base + 80 + L, L)]
            g = (jnp.where(lo8, _vgather(t0, idx_e), _vgather(t1, idx_e))
                 + jnp.where(lo8, _vgather(t0, idx_o), _vgather(t1, idx_o)))
            rc = (((ha + dy - 1 >= 0) & (ha + dy - 1 < H)).astype(jnp.float32)
                  + ((ha + dy >= 0) & (ha + dy < H)).astype(jnp.float32))
            if dx == 0:
                colc = jnp.where(wbv == 0, 1.0, 2.0)
            elif dx == 2:
                colc = jnp.where(wbv == HB - 1, 1.0, 2.0)
            else:
                colc = jnp.full((L,), 2.0, jnp.float32)
            Pv.append(g)
            Nv.append(rc * colc - g)
        sf = [ori_v[pl.ds(kk * 64 + hb * 32 + w0, L)] for kk in range(KK)]
        maskf = jnp.where(sf[0] > 0, 1.0, 0.0)
        one16 = zero16 + 1.0

        def select(payload, nsel, largest):
            alive = [one16 for _ in range(KK)]
            tot = zero16
            for _s in range(nsel):
                cur = neg_inf if largest else pos_inf
                curp = zero16
                for kk in range(KK):
                    if largest:
                        better = jnp.where(sf[kk] > cur, 1.0, 0.0)
                    else:
                        better = jnp.where(sf[kk] < cur, 1.0, 0.0)
                    take = alive[kk] * better
                    cur = jnp.where(take > 0.5, sf[kk], cur)
                    curp = jnp.where(take > 0.5, payload[kk], curp)
                fm = zero16
                for kk in range(KK):
                    eq = jnp.where(sf[kk] == cur, 1.0, 0.0)
                    selm = alive[kk] * eq * (1.0 - fm)
                    fm = fm + selm
                    alive[kk] = alive[kk] - selm
                if largest:
                    tot = tot - cur * curp
                else:
                    tot = tot - (1.0 - cur) * curp
            return tot
        possum = select(Pv, 5, True)
        negsum = select(Nv, 4, False)
        accp = accp + maskf * possum
        accn = accn + maskf * negsum
        accc = accc + maskf
        return accp, accn, accc

    accp, accn, accc = lax.fori_loop(0, 4, _block, (zero16, zero16, zero16))

    def _st(i, _):
        off = i * L
        s = zero16
        for kk in range(KK):
            s = s + ori_v[pl.ds(kk * 64 + off, L)]
        state_v[pl.ds(off, L)] = s * (1.0 / KK)
        return 0
    lax.fori_loop(0, 4, _st, 0)
    soff = pl.multiple_of(b * (HB * HB) + jb0 * HB, 8)
    pltpu.sync_copy(state_v, state_hbm.at[pl.ds(soff, 2 * HB)])

    out_v[pl.ds(0, L)] = accp
    out_v[pl.ds(L, L)] = accn
    out_v[pl.ds(2 * L, L)] = accc
    nw = NC * NS
    for f in range(3):
        pltpu.sync_copy(out_v.at[pl.ds(f * L, L)],
                        part_hbm.at[pl.ds(f * nw * L + wid * L, L)])


def _tc_combine(part_ref, state_ref, lp_ref, ln_ref, state_out_ref):
    nw = NC * NS
    sp = jnp.sum(part_ref[pl.ds(0, nw * L)])
    sn = jnp.sum(part_ref[pl.ds(nw * L, nw * L)])
    c32 = jnp.sum(part_ref[pl.ds(2 * nw * L, nw * L)])
    cnt_pos = jnp.maximum(c32 * 4.0 * 5.0, 1.0)
    cnt_neg = jnp.maximum(c32 * 4.0 * 4.0, 1.0)
    lp_ref[0] = sp / cnt_pos * 0.5
    ln_ref[0] = sn / cnt_neg
    for b in range(B):
        for r in range(HB):
            flat = b * HB * HB + r * HB
            state_out_ref[b, r, :] = state_ref[flat // 128,
                                               pl.ds(flat % 128, HB)]


@functools.partial(
    pl.kernel,
    mesh=plsc.VectorSubcoreMesh(core_axis_name="c", subcore_axis_name="s"),
    out_type=[
        jax.ShapeDtypeStruct((NC * NS * 3 * L,), jnp.float32),
        jax.ShapeDtypeStruct((B * HB * HB,), jnp.float32),
    ],
    scratch_types=[
        pltpu.VMEM((C * 6 * W,), jnp.float32),
        pltpu.VMEM((C * 6 * 80,), jnp.float32),
        pltpu.VMEM((5 * 5 * 80,), jnp.float32),
        pltpu.VMEM((KK * 2 * HB,), jnp.float32),
        pltpu.VMEM((2 * HB,), jnp.float32),
        pltpu.VMEM((3 * L,), jnp.float32),
        pltpu.SemaphoreType.DMA,
        pltpu.SemaphoreType.DMA,
    ],
)
def _sc_kernel(seg_hbm, ori_hbm, part_hbm, state_hbm, *scratch):
    _sc_body(seg_hbm, ori_hbm, part_hbm, state_hbm, *scratch)


def kernel(ori_sim_feats_list, seg_logits):
    partials, state = _sc_kernel(seg_logits.reshape(-1),
                                 ori_sim_feats_list.reshape(-1))
    lp, ln, state3 = pl.pallas_call(
        _tc_combine,
        out_shape=[jax.ShapeDtypeStruct((1,), jnp.float32),
                   jax.ShapeDtypeStruct((1,), jnp.float32),
                   jax.ShapeDtypeStruct((B, HB, HB), jnp.float32)],
        out_specs=[pl.BlockSpec(memory_space=pltpu.SMEM),
                   pl.BlockSpec(memory_space=pltpu.SMEM),
                   pl.BlockSpec(memory_space=pltpu.VMEM)],
    )(partials, state.reshape(16, 128))
    return lp.reshape(()), ln.reshape(()), state3
